# baseline (speedup 1.0000x reference)
import jax
import jax.numpy as jnp
from jax import lax
from jax.experimental import pallas as pl
from jax.experimental.pallas import tpu as pltpu
from jax.experimental.pallas import tpu_sc as plsc

N = 10000
E = 320000
D = 128
DH = D // 2

NC = 2
NS = 16
EPS = E // NS
K = 125
NCHUNK = EPS // K
IB = 40
NIB = NCHUNK // IB
NBUF = 4
LAG = 2
ZT = 10
RPT = N // ZT
ZCH = 128
CW = 16


def _fill_vmem_rows(ref, nrows, width, vec16):
    def fill(i, carry):
        for j in range(width // 16):
            ref[i, pl.ds(j * 16, 16)] = vec16
        return carry
    lax.fori_loop(0, nrows, fill, 0)


def _zero_sh_slice(zbuf, sh, base):
    for k in range(7):
        pltpu.sync_copy(zbuf, sh.at[pl.ds(base + k * ZCH, ZCH)])
    pltpu.sync_copy(zbuf.at[pl.ds(0, 104)], sh.at[pl.ds(base + 896, 104)])


def _agg_common(xa, xb, edges, out_sum,
                srcb, dstb, bufs, zrows, acc_sh, sems, ssems, isem,
                chunk_extra):
    c = lax.axis_index("c")
    s = lax.axis_index("s")

    _fill_vmem_rows(zrows, ZCH, DH, jnp.zeros((16,), jnp.float32))

    base = s * RPT

    @pl.when(s < ZT)
    def _zero():
        _zero_sh_slice(zrows, acc_sh, base)

    pltpu.sync_copy(edges.at[0, s, pl.ds(0, IB)], srcb.at[0])
    pltpu.sync_copy(edges.at[1, s, pl.ds(0, IB)], dstb.at[0])

    plsc.subcore_barrier()

    def start_gather(jb, i, buf, sem):
        @pl.when(c == 0)
        def _g0():
            pltpu.async_copy(xa.at[srcb.at[jb, i]], buf, sem)

        @pl.when(c == 1)
        def _g1():
            pltpu.async_copy(xb.at[srcb.at[jb, i]], buf, sem)

    for j in range(NIB):
        jb = j % 2
        nb = (j + 1) % 2
        if j + 1 < NIB:
            pltpu.async_copy(edges.at[0, s, pl.ds((j + 1) * IB, IB)],
                             srcb.at[nb], isem)
            pltpu.async_copy(edges.at[1, s, pl.ds((j + 1) * IB, IB)],
                             dstb.at[nb], isem)

        nbuf = len(bufs)
        for b in range(nbuf):
            start_gather(jb, b, bufs[b], sems[b])

        for i in range(IB):
            b = i % nbuf
            buf = bufs[b]
            pltpu.make_async_copy(xa.at[srcb.at[jb, i]], buf, sems[b]).wait()
            pltpu.async_copy(buf, acc_sh.at[dstb.at[jb, i]], ssems[b],
                             add=True)
            chunk_extra(c, jb, i, dstb)
            nxt = i + LAG
            if nxt < IB and nxt >= nbuf:
                b2 = nxt % nbuf
                prev = nxt - nbuf
                pltpu.make_async_copy(bufs[b2],
                                      acc_sh.at[dstb.at[jb, prev]],
                                      ssems[b2]).wait()
                start_gather(jb, nxt, bufs[b2], sems[b2])

        for t in range(nbuf):
            i2 = IB - nbuf + t
            b2 = i2 % nbuf
            pltpu.make_async_copy(bufs[b2],
                                  acc_sh.at[dstb.at[jb, i2]],
                                  ssems[b2]).wait()

        if j + 1 < NIB:
            pltpu.make_async_copy(edges.at[0, s, pl.ds((j + 1) * IB, IB)],
                                  srcb.at[nb], isem).wait()
            pltpu.make_async_copy(edges.at[1, s, pl.ds((j + 1) * IB, IB)],
                                  dstb.at[nb], isem).wait()

    plsc.subcore_barrier()

    @pl.when(s < ZT)
    def _copyout():
        pltpu.sync_copy(acc_sh.at[pl.ds(base, RPT)],
                        out_sum.at[c, pl.ds(base, RPT)])


def _sc_agg_cnt_body(xa, xb, edges, out_sum, out_cnt,
                     srcb, dstb, rows0, rows1, rows2, rows3,
                     zrows, ones_v, zcnt,
                     acc_sh, cnt_sh, sem0, sem1, sem2, sem3,
                     ssem0, ssem1, ssem2, ssem3, isem):
    s = lax.axis_index("s")
    base = s * RPT

    _fill_vmem_rows(zcnt, ZCH, CW, jnp.zeros((16,), jnp.float32))
    _fill_vmem_rows(ones_v, K, CW, jnp.ones((16,), jnp.float32))

    @pl.when(s < ZT)
    def _zero_cnt():
        _zero_sh_slice(zcnt, cnt_sh, base)

    def chunk_extra(c, jb, i, dstb):
        @pl.when(c == i % 2)
        def _cnt():
            pltpu.sync_copy(ones_v, cnt_sh.at[dstb.at[jb, i]], add=True)

    _agg_common(xa, xb, edges, out_sum,
                srcb, dstb, (rows0, rows1, rows2, rows3), zrows, acc_sh,
                (sem0, sem1, sem2, sem3),
                (ssem0, ssem1, ssem2, ssem3), isem, chunk_extra)

    c = lax.axis_index("c")

    @pl.when(s < ZT)
    def _copyout_cnt():
        pltpu.sync_copy(cnt_sh.at[pl.ds(base, RPT)],
                        out_cnt.at[c, pl.ds(base, RPT)])


def _sc_agg_body(xa, xb, edges, out_sum,
                 srcb, dstb, rows0, rows1, rows2, rows3, zrows, acc_sh,
                 sem0, sem1, sem2, sem3,
                 ssem0, ssem1, ssem2, ssem3, isem):
    _agg_common(xa, xb, edges, out_sum,
                srcb, dstb, (rows0, rows1, rows2, rows3), zrows, acc_sh,
                (sem0, sem1, sem2, sem3),
                (ssem0, ssem1, ssem2, ssem3), isem,
                lambda c, jb, i, dstb: None)


_MESH = dict(core_axis_name="c", subcore_axis_name="s")


def _sc_agg_cnt(xa, xb, edges):
    return pl.kernel(
        _sc_agg_cnt_body,
        out_type=[
            jax.ShapeDtypeStruct((NC, N, DH), jnp.float32),
            jax.ShapeDtypeStruct((NC, N, CW), jnp.float32),
        ],
        mesh=plsc.VectorSubcoreMesh(**_MESH),
        compiler_params=pltpu.CompilerParams(use_tc_tiling_on_sc=False),
        scratch_types=[
            pltpu.VMEM((2, IB, K), jnp.int32),
            pltpu.VMEM((2, IB, K), jnp.int32),
            pltpu.VMEM((K, DH), jnp.float32),
            pltpu.VMEM((K, DH), jnp.float32),
            pltpu.VMEM((K, DH), jnp.float32),
            pltpu.VMEM((K, DH), jnp.float32),
            pltpu.VMEM((ZCH, DH), jnp.float32),
            pltpu.VMEM((K, CW), jnp.float32),
            pltpu.VMEM((ZCH, CW), jnp.float32),
            pltpu.VMEM_SHARED((N, DH), jnp.float32),
            pltpu.VMEM_SHARED((N, CW), jnp.float32),
            pltpu.SemaphoreType.DMA,
            pltpu.SemaphoreType.DMA,
            pltpu.SemaphoreType.DMA,
            pltpu.SemaphoreType.DMA,
            pltpu.SemaphoreType.DMA,
            pltpu.SemaphoreType.DMA,
            pltpu.SemaphoreType.DMA,
            pltpu.SemaphoreType.DMA,
            pltpu.SemaphoreType.DMA,
        ],
    )(xa, xb, edges)


def _sc_agg(xa, xb, edges):
    return pl.kernel(
        _sc_agg_body,
        out_type=jax.ShapeDtypeStruct((NC, N, DH), jnp.float32),
        mesh=plsc.VectorSubcoreMesh(**_MESH),
        compiler_params=pltpu.CompilerParams(use_tc_tiling_on_sc=False),
        scratch_types=[
            pltpu.VMEM((2, IB, K), jnp.int32),
            pltpu.VMEM((2, IB, K), jnp.int32),
            pltpu.VMEM((K, DH), jnp.float32),
            pltpu.VMEM((K, DH), jnp.float32),
            pltpu.VMEM((K, DH), jnp.float32),
            pltpu.VMEM((K, DH), jnp.float32),
            pltpu.VMEM((ZCH, DH), jnp.float32),
            pltpu.VMEM_SHARED((N, DH), jnp.float32),
            pltpu.SemaphoreType.DMA,
            pltpu.SemaphoreType.DMA,
            pltpu.SemaphoreType.DMA,
            pltpu.SemaphoreType.DMA,
            pltpu.SemaphoreType.DMA,
            pltpu.SemaphoreType.DMA,
            pltpu.SemaphoreType.DMA,
            pltpu.SemaphoreType.DMA,
            pltpu.SemaphoreType.DMA,
        ],
    )(xa, xb, edges)


BN = 2000

_DNT = (((1,), (1,)), ((), ()))


def _matmul_t(a, w):
    return lax.dot_general(a, w, _DNT, preferred_element_type=jnp.float32)


def _tc_pre1_body(x_ref, wr_ref, b_ref, r_ref, xa_ref, xb_ref):
    x = x_ref[...]
    r_ref[...] = _matmul_t(x, wr_ref[...]) + b_ref[...]
    xa_ref[...] = x[:, :DH]
    xb_ref[...] = x[:, DH:]


def _tc_pre1(x, wr, b):
    return pl.pallas_call(
        _tc_pre1_body,
        grid=(N // BN,),
        in_specs=[
            pl.BlockSpec((BN, D), lambda i: (i, 0)),
            pl.BlockSpec((D, D), lambda i: (0, 0)),
            pl.BlockSpec((1, D), lambda i: (0, 0)),
        ],
        out_specs=[
            pl.BlockSpec((BN, D), lambda i: (i, 0)),
            pl.BlockSpec((BN, DH), lambda i: (i, 0)),
            pl.BlockSpec((BN, DH), lambda i: (i, 0)),
        ],
        out_shape=[
            jax.ShapeDtypeStruct((N, D), jnp.float32),
            jax.ShapeDtypeStruct((N, DH), jnp.float32),
            jax.ShapeDtypeStruct((N, DH), jnp.float32),
        ],
    )(x, wr, b)


def _tc_post1_body(sum_ref, cnt_ref, r_ref, wl_ref, wr2_ref, b2_ref,
                   ha_ref, hb_ref, r2_ref):
    ssum = jnp.concatenate([sum_ref[0], sum_ref[1]], axis=1)
    cnt = cnt_ref[0, :, 0:1] + cnt_ref[1, :, 0:1]
    inv = 1.0 / jnp.maximum(cnt, 1.0)
    mean = ssum * inv
    h = jnp.maximum(_matmul_t(mean, wl_ref[...]) + r_ref[...], 0.0)
    ha_ref[...] = h[:, :DH]
    hb_ref[...] = h[:, DH:]
    r2_ref[...] = _matmul_t(h, wr2_ref[...]) + b2_ref[...]


def _tc_post1(sum_p, cnt_p, r1, wl, wr2, b2):
    return pl.pallas_call(
        _tc_post1_body,
        grid=(N // BN,),
        in_specs=[
            pl.BlockSpec((NC, BN, DH), lambda i: (0, i, 0)),
            pl.BlockSpec((NC, BN, CW), lambda i: (0, i, 0)),
            pl.BlockSpec((BN, D), lambda i: (i, 0)),
            pl.BlockSpec((D, D), lambda i: (0, 0)),
            pl.BlockSpec((D, D), lambda i: (0, 0)),
            pl.BlockSpec((1, D), lambda i: (0, 0)),
        ],
        out_specs=[
            pl.BlockSpec((BN, DH), lambda i: (i, 0)),
            pl.BlockSpec((BN, DH), lambda i: (i, 0)),
            pl.BlockSpec((BN, D), lambda i: (i, 0)),
        ],
        out_shape=[
            jax.ShapeDtypeStruct((N, DH), jnp.float32),
            jax.ShapeDtypeStruct((N, DH), jnp.float32),
            jax.ShapeDtypeStruct((N, D), jnp.float32),
        ],
    )(sum_p, cnt_p, r1, wl, wr2, b2)


def _tc_post2_body(sum_ref, cnt_ref, r2_ref, wl_ref, wfc_ref, bfc_ref,
                   out_ref):
    ssum = jnp.concatenate([sum_ref[0], sum_ref[1]], axis=1)
    cnt = cnt_ref[0, :, 0:1] + cnt_ref[1, :, 0:1]
    inv = 1.0 / jnp.maximum(cnt, 1.0)
    mean = ssum * inv
    h2 = jnp.maximum(_matmul_t(mean, wl_ref[...]) + r2_ref[...], 0.0)
    logit = jnp.sum(h2 * wfc_ref[...], axis=1, keepdims=True)
    out_ref[...] = jax.nn.sigmoid(logit + bfc_ref[...])


def _tc_post2(sum_p, cnt_p, r2, wl, wfc, bfc):
    return pl.pallas_call(
        _tc_post2_body,
        grid=(N // BN,),
        in_specs=[
            pl.BlockSpec((NC, BN, DH), lambda i: (0, i, 0)),
            pl.BlockSpec((NC, BN, CW), lambda i: (0, i, 0)),
            pl.BlockSpec((BN, D), lambda i: (i, 0)),
            pl.BlockSpec((D, D), lambda i: (0, 0)),
            pl.BlockSpec((1, D), lambda i: (0, 0)),
            pl.BlockSpec((1, 1), lambda i: (0, 0)),
        ],
        out_specs=pl.BlockSpec((BN, 1), lambda i: (i, 0)),
        out_shape=jax.ShapeDtypeStruct((N, 1), jnp.float32),
    )(sum_p, cnt_p, r2, wl, wfc, bfc)


@jax.jit
def kernel(x, edge_index, W1_l, b1, W1_r, W2_l, b2, W2_r, Wfc, bfc):
    edges = edge_index.astype(jnp.int32).reshape(2, NS, NCHUNK, K)

    r1, xa, xb = _tc_pre1(x, W1_r, b1.reshape(1, D))
    sum1, cnt = _sc_agg_cnt(xa, xb, edges)
    ha, hb, r2 = _tc_post1(sum1, cnt, r1, W1_l, W2_r, b2.reshape(1, D))
    sum2 = _sc_agg(ha, hb, edges)
    out = _tc_post2(sum2, cnt, r2, W2_l, Wfc, bfc.reshape(1, 1))
    return out


# speedup vs baseline: 1.1823x; 1.1823x over previous
import jax
import jax.numpy as jnp
from jax import lax
from jax.experimental import pallas as pl
from jax.experimental.pallas import tpu as pltpu
from jax.experimental.pallas import tpu_sc as plsc

N = 10000
E = 320000
D = 128
DH = D // 2

NC = 2
NS = 16
EPS = E // NS
K = 125
NCHUNK = EPS // K
IB = 40
NIB = NCHUNK // IB
NBUF = 3
ZT = 10
RPT = N // ZT
ZCH = 128
CW = 16


def _fill_vmem_rows(ref, nrows, width, vec16):
    def fill(i, carry):
        for j in range(width // 16):
            ref[i, pl.ds(j * 16, 16)] = vec16
        return carry
    lax.fori_loop(0, nrows, fill, 0)


def _zero_sh_slice(zbuf, sh, base):
    for k in range(7):
        pltpu.sync_copy(zbuf, sh.at[pl.ds(base + k * ZCH, ZCH)])
    pltpu.sync_copy(zbuf.at[pl.ds(0, 104)], sh.at[pl.ds(base + 896, 104)])


def _agg_common(xa, xb, edges, out_sum,
                srcb, dstb, bufs, zrows, acc_sh, sems, isem,
                chunk_extra):
    c = lax.axis_index("c")
    s = lax.axis_index("s")

    _fill_vmem_rows(zrows, ZCH, DH, jnp.zeros((16,), jnp.float32))

    base = s * RPT

    @pl.when(s < ZT)
    def _zero():
        _zero_sh_slice(zrows, acc_sh, base)

    pltpu.sync_copy(edges.at[0, s, pl.ds(0, IB)], srcb.at[0])
    pltpu.sync_copy(edges.at[1, s, pl.ds(0, IB)], dstb.at[0])

    plsc.subcore_barrier()

    def start_gather(jb, i, buf, sem):
        @pl.when(c == 0)
        def _g0():
            pltpu.async_copy(xa.at[srcb.at[jb, i]], buf, sem)

        @pl.when(c == 1)
        def _g1():
            pltpu.async_copy(xb.at[srcb.at[jb, i]], buf, sem)

    for j in range(NIB):
        jb = j % 2
        nb = (j + 1) % 2
        if j + 1 < NIB:
            pltpu.async_copy(edges.at[0, s, pl.ds((j + 1) * IB, IB)],
                             srcb.at[nb], isem)
            pltpu.async_copy(edges.at[1, s, pl.ds((j + 1) * IB, IB)],
                             dstb.at[nb], isem)

        nbuf = len(bufs)
        for b in range(nbuf):
            start_gather(jb, b, bufs[b], sems[b])

        for i in range(IB):
            b = i % nbuf
            buf, sem = bufs[b], sems[b]
            pltpu.make_async_copy(xa.at[srcb.at[jb, i]], buf, sem).wait()
            pltpu.sync_copy(buf, acc_sh.at[dstb.at[jb, i]], add=True)
            chunk_extra(c, jb, i, dstb)
            nxt = i + nbuf
            if nxt < IB:
                start_gather(jb, nxt, bufs[b], sems[b])

        if j + 1 < NIB:
            pltpu.make_async_copy(edges.at[0, s, pl.ds((j + 1) * IB, IB)],
                                  srcb.at[nb], isem).wait()
            pltpu.make_async_copy(edges.at[1, s, pl.ds((j + 1) * IB, IB)],
                                  dstb.at[nb], isem).wait()

    plsc.subcore_barrier()

    @pl.when(s < ZT)
    def _copyout():
        pltpu.sync_copy(acc_sh.at[pl.ds(base, RPT)],
                        out_sum.at[c, pl.ds(base, RPT)])


def _sc_agg_cnt_body(xa, xb, edges, out_sum, out_cnt,
                     srcb, dstb, rows0, rows1, rows2, rows3,
                     zrows, ones_v, zcnt,
                     acc_sh, cnt_sh, sem0, sem1, sem2, sem3, isem):
    s = lax.axis_index("s")
    base = s * RPT

    _fill_vmem_rows(zcnt, ZCH, CW, jnp.zeros((16,), jnp.float32))
    _fill_vmem_rows(ones_v, K, CW, jnp.ones((16,), jnp.float32))

    @pl.when(s < ZT)
    def _zero_cnt():
        _zero_sh_slice(zcnt, cnt_sh, base)

    def chunk_extra(c, jb, i, dstb):
        @pl.when(c == i % 2)
        def _cnt():
            pltpu.sync_copy(ones_v, cnt_sh.at[dstb.at[jb, i]], add=True)

    _agg_common(xa, xb, edges, out_sum,
                srcb, dstb, (rows0, rows1, rows2, rows3), zrows, acc_sh,
                (sem0, sem1, sem2, sem3), isem, chunk_extra)

    c = lax.axis_index("c")

    @pl.when(s < ZT)
    def _copyout_cnt():
        pltpu.sync_copy(cnt_sh.at[pl.ds(base, RPT)],
                        out_cnt.at[c, pl.ds(base, RPT)])


def _sc_agg_body(xa, xb, edges, out_sum,
                 srcb, dstb, rows0, rows1, rows2, rows3, zrows, acc_sh,
                 sem0, sem1, sem2, sem3, isem):
    _agg_common(xa, xb, edges, out_sum,
                srcb, dstb, (rows0, rows1, rows2, rows3), zrows, acc_sh,
                (sem0, sem1, sem2, sem3), isem, lambda c, jb, i, dstb: None)


_MESH = dict(core_axis_name="c", subcore_axis_name="s")


def _sc_agg_cnt(xa, xb, edges):
    return pl.kernel(
        _sc_agg_cnt_body,
        out_type=[
            jax.ShapeDtypeStruct((NC, N, DH), jnp.float32),
            jax.ShapeDtypeStruct((NC, N, CW), jnp.float32),
        ],
        mesh=plsc.VectorSubcoreMesh(**_MESH),
        compiler_params=pltpu.CompilerParams(use_tc_tiling_on_sc=False),
        scratch_types=[
            pltpu.VMEM((2, IB, K), jnp.int32),
            pltpu.VMEM((2, IB, K), jnp.int32),
            pltpu.VMEM((K, DH), jnp.float32),
            pltpu.VMEM((K, DH), jnp.float32),
            pltpu.VMEM((K, DH), jnp.float32),
            pltpu.VMEM((K, DH), jnp.float32),
            pltpu.VMEM((ZCH, DH), jnp.float32),
            pltpu.VMEM((K, CW), jnp.float32),
            pltpu.VMEM((ZCH, CW), jnp.float32),
            pltpu.VMEM_SHARED((N, DH), jnp.float32),
            pltpu.VMEM_SHARED((N, CW), jnp.float32),
            pltpu.SemaphoreType.DMA,
            pltpu.SemaphoreType.DMA,
            pltpu.SemaphoreType.DMA,
            pltpu.SemaphoreType.DMA,
            pltpu.SemaphoreType.DMA,
        ],
    )(xa, xb, edges)


def _sc_agg(xa, xb, edges):
    return pl.kernel(
        _sc_agg_body,
        out_type=jax.ShapeDtypeStruct((NC, N, DH), jnp.float32),
        mesh=plsc.VectorSubcoreMesh(**_MESH),
        compiler_params=pltpu.CompilerParams(use_tc_tiling_on_sc=False),
        scratch_types=[
            pltpu.VMEM((2, IB, K), jnp.int32),
            pltpu.VMEM((2, IB, K), jnp.int32),
            pltpu.VMEM((K, DH), jnp.float32),
            pltpu.VMEM((K, DH), jnp.float32),
            pltpu.VMEM((K, DH), jnp.float32),
            pltpu.VMEM((K, DH), jnp.float32),
            pltpu.VMEM((ZCH, DH), jnp.float32),
            pltpu.VMEM_SHARED((N, DH), jnp.float32),
            pltpu.SemaphoreType.DMA,
            pltpu.SemaphoreType.DMA,
            pltpu.SemaphoreType.DMA,
            pltpu.SemaphoreType.DMA,
            pltpu.SemaphoreType.DMA,
        ],
    )(xa, xb, edges)


BN = 2000

_DNT = (((1,), (1,)), ((), ()))


def _matmul_t(a, w):
    return lax.dot_general(a, w, _DNT, preferred_element_type=jnp.float32)


def _tc_pre1_body(x_ref, wr_ref, b_ref, r_ref, xa_ref, xb_ref):
    x = x_ref[...]
    r_ref[...] = _matmul_t(x, wr_ref[...]) + b_ref[...]
    xa_ref[...] = x[:, :DH]
    xb_ref[...] = x[:, DH:]


def _tc_pre1(x, wr, b):
    return pl.pallas_call(
        _tc_pre1_body,
        grid=(N // BN,),
        in_specs=[
            pl.BlockSpec((BN, D), lambda i: (i, 0)),
            pl.BlockSpec((D, D), lambda i: (0, 0)),
            pl.BlockSpec((1, D), lambda i: (0, 0)),
        ],
        out_specs=[
            pl.BlockSpec((BN, D), lambda i: (i, 0)),
            pl.BlockSpec((BN, DH), lambda i: (i, 0)),
            pl.BlockSpec((BN, DH), lambda i: (i, 0)),
        ],
        out_shape=[
            jax.ShapeDtypeStruct((N, D), jnp.float32),
            jax.ShapeDtypeStruct((N, DH), jnp.float32),
            jax.ShapeDtypeStruct((N, DH), jnp.float32),
        ],
    )(x, wr, b)


def _tc_post1_body(sum_ref, cnt_ref, r_ref, wl_ref, wr2_ref, b2_ref,
                   ha_ref, hb_ref, r2_ref):
    ssum = jnp.concatenate([sum_ref[0], sum_ref[1]], axis=1)
    cnt = cnt_ref[0, :, 0:1] + cnt_ref[1, :, 0:1]
    inv = 1.0 / jnp.maximum(cnt, 1.0)
    mean = ssum * inv
    h = jnp.maximum(_matmul_t(mean, wl_ref[...]) + r_ref[...], 0.0)
    ha_ref[...] = h[:, :DH]
    hb_ref[...] = h[:, DH:]
    r2_ref[...] = _matmul_t(h, wr2_ref[...]) + b2_ref[...]


def _tc_post1(sum_p, cnt_p, r1, wl, wr2, b2):
    return pl.pallas_call(
        _tc_post1_body,
        grid=(N // BN,),
        in_specs=[
            pl.BlockSpec((NC, BN, DH), lambda i: (0, i, 0)),
            pl.BlockSpec((NC, BN, CW), lambda i: (0, i, 0)),
            pl.BlockSpec((BN, D), lambda i: (i, 0)),
            pl.BlockSpec((D, D), lambda i: (0, 0)),
            pl.BlockSpec((D, D), lambda i: (0, 0)),
            pl.BlockSpec((1, D), lambda i: (0, 0)),
        ],
        out_specs=[
            pl.BlockSpec((BN, DH), lambda i: (i, 0)),
            pl.BlockSpec((BN, DH), lambda i: (i, 0)),
            pl.BlockSpec((BN, D), lambda i: (i, 0)),
        ],
        out_shape=[
            jax.ShapeDtypeStruct((N, DH), jnp.float32),
            jax.ShapeDtypeStruct((N, DH), jnp.float32),
            jax.ShapeDtypeStruct((N, D), jnp.float32),
        ],
    )(sum_p, cnt_p, r1, wl, wr2, b2)


def _tc_post2_body(sum_ref, cnt_ref, r2_ref, wl_ref, wfc_ref, bfc_ref,
                   out_ref):
    ssum = jnp.concatenate([sum_ref[0], sum_ref[1]], axis=1)
    cnt = cnt_ref[0, :, 0:1] + cnt_ref[1, :, 0:1]
    inv = 1.0 / jnp.maximum(cnt, 1.0)
    mean = ssum * inv
    h2 = jnp.maximum(_matmul_t(mean, wl_ref[...]) + r2_ref[...], 0.0)
    logit = jnp.sum(h2 * wfc_ref[...], axis=1, keepdims=True)
    out_ref[...] = jax.nn.sigmoid(logit + bfc_ref[...])


def _tc_post2(sum_p, cnt_p, r2, wl, wfc, bfc):
    return pl.pallas_call(
        _tc_post2_body,
        grid=(N // BN,),
        in_specs=[
            pl.BlockSpec((NC, BN, DH), lambda i: (0, i, 0)),
            pl.BlockSpec((NC, BN, CW), lambda i: (0, i, 0)),
            pl.BlockSpec((BN, D), lambda i: (i, 0)),
            pl.BlockSpec((D, D), lambda i: (0, 0)),
            pl.BlockSpec((1, D), lambda i: (0, 0)),
            pl.BlockSpec((1, 1), lambda i: (0, 0)),
        ],
        out_specs=pl.BlockSpec((BN, 1), lambda i: (i, 0)),
        out_shape=jax.ShapeDtypeStruct((N, 1), jnp.float32),
    )(sum_p, cnt_p, r2, wl, wfc, bfc)


@jax.jit
def kernel(x, edge_index, W1_l, b1, W1_r, W2_l, b2, W2_r, Wfc, bfc):
    edges = edge_index.astype(jnp.int32).reshape(2, NS, NCHUNK, K)

    r1, xa, xb = _tc_pre1(x, W1_r, b1.reshape(1, D))
    sum1, cnt = _sc_agg_cnt(xa, xb, edges)
    ha, hb, r2 = _tc_post1(sum1, cnt, r1, W1_l, W2_r, b2.reshape(1, D))
    sum2 = _sc_agg(ha, hb, edges)
    out = _tc_post2(sum2, cnt, r2, W2_l, Wfc, bfc.reshape(1, 1))
    return out


# speedup vs baseline: 1.1896x; 1.0062x over previous
import jax
import jax.numpy as jnp
from jax import lax
from jax.experimental import pallas as pl
from jax.experimental.pallas import tpu as pltpu
from jax.experimental.pallas import tpu_sc as plsc

N = 10000
E = 320000
D = 128
DH = D // 2

NC = 2
NS = 16
EPS = E // NS
K = 125
NCHUNK = EPS // K
IB = 40
NIB = NCHUNK // IB
NBUF = 3
ZT = 10
RPT = N // ZT
ZCH = 128
CW = 16


def _fill_vmem_rows(ref, nrows, width, vec16):
    def fill(i, carry):
        for j in range(width // 16):
            ref[i, pl.ds(j * 16, 16)] = vec16
        return carry
    lax.fori_loop(0, nrows, fill, 0)


def _zero_sh_slice(zbuf, sh, base):
    for k in range(7):
        pltpu.sync_copy(zbuf, sh.at[pl.ds(base + k * ZCH, ZCH)])
    pltpu.sync_copy(zbuf.at[pl.ds(0, 104)], sh.at[pl.ds(base + 896, 104)])


def _agg_common(xa, xb, edges, out_sum,
                srcb, dstb, bufs, zrows, acc_sh, sems, isem,
                chunk_extra):
    c = lax.axis_index("c")
    s = lax.axis_index("s")

    _fill_vmem_rows(zrows, ZCH, DH, jnp.zeros((16,), jnp.float32))

    base = s * RPT

    @pl.when(s < ZT)
    def _zero():
        _zero_sh_slice(zrows, acc_sh, base)

    pltpu.sync_copy(edges.at[0, s, pl.ds(0, IB)], srcb.at[0])
    pltpu.sync_copy(edges.at[1, s, pl.ds(0, IB)], dstb.at[0])

    plsc.subcore_barrier()

    def start_gather(jb, i, buf, sem):
        @pl.when(c == 0)
        def _g0():
            pltpu.async_copy(xa.at[srcb.at[jb, i]], buf, sem)

        @pl.when(c == 1)
        def _g1():
            pltpu.async_copy(xb.at[srcb.at[jb, i]], buf, sem)

    for j in range(NIB):
        jb = j % 2
        nb = (j + 1) % 2
        if j + 1 < NIB:
            pltpu.async_copy(edges.at[0, s, pl.ds((j + 1) * IB, IB)],
                             srcb.at[nb], isem)
            pltpu.async_copy(edges.at[1, s, pl.ds((j + 1) * IB, IB)],
                             dstb.at[nb], isem)

        nbuf = len(bufs)
        for b in range(nbuf):
            start_gather(jb, b, bufs[b], sems[b])

        for i in range(IB):
            b = i % nbuf
            buf, sem = bufs[b], sems[b]
            pltpu.make_async_copy(xa.at[srcb.at[jb, i]], buf, sem).wait()
            pltpu.sync_copy(buf, acc_sh.at[dstb.at[jb, i]], add=True)
            chunk_extra(c, jb, i, dstb)
            nxt = i + nbuf
            if nxt < IB:
                start_gather(jb, nxt, bufs[b], sems[b])

        if j + 1 < NIB:
            pltpu.make_async_copy(edges.at[0, s, pl.ds((j + 1) * IB, IB)],
                                  srcb.at[nb], isem).wait()
            pltpu.make_async_copy(edges.at[1, s, pl.ds((j + 1) * IB, IB)],
                                  dstb.at[nb], isem).wait()

    plsc.subcore_barrier()

    @pl.when(s < ZT)
    def _copyout():
        pltpu.sync_copy(acc_sh.at[pl.ds(base, RPT)],
                        out_sum.at[c, pl.ds(base, RPT)])


def _sc_agg_cnt_body(xa, xb, edges, out_sum, out_cnt,
                     srcb, dstb, rows0, rows1, rows2, rows3,
                     zrows, ones_v, zcnt,
                     acc_sh, cnt_sh, sem0, sem1, sem2, sem3, isem):
    s = lax.axis_index("s")
    base = s * RPT

    _fill_vmem_rows(zcnt, ZCH, CW, jnp.zeros((16,), jnp.float32))
    _fill_vmem_rows(ones_v, K, CW, jnp.ones((16,), jnp.float32))

    @pl.when(s < ZT)
    def _zero_cnt():
        _zero_sh_slice(zcnt, cnt_sh, base)

    def chunk_extra(c, jb, i, dstb):
        @pl.when(c == i % 2)
        def _cnt():
            pltpu.sync_copy(ones_v, cnt_sh.at[dstb.at[jb, i]], add=True)

    _agg_common(xa, xb, edges, out_sum,
                srcb, dstb, (rows0, rows1, rows2, rows3), zrows, acc_sh,
                (sem0, sem1, sem2, sem3), isem, chunk_extra)

    c = lax.axis_index("c")

    @pl.when(s < ZT)
    def _copyout_cnt():
        pltpu.sync_copy(cnt_sh.at[pl.ds(base, RPT)],
                        out_cnt.at[c, pl.ds(base, RPT)])


def _sc_agg_body(xa, xb, edges, out_sum,
                 srcb, dstb, rows0, rows1, rows2, rows3, zrows, acc_sh,
                 sem0, sem1, sem2, sem3, isem):
    _agg_common(xa, xb, edges, out_sum,
                srcb, dstb, (rows0, rows1, rows2, rows3), zrows, acc_sh,
                (sem0, sem1, sem2, sem3), isem, lambda c, jb, i, dstb: None)


_MESH = dict(core_axis_name="c", subcore_axis_name="s")


def _sc_agg_cnt(xa, xb, edges):
    return pl.kernel(
        _sc_agg_cnt_body,
        out_type=[
            jax.ShapeDtypeStruct((NC, N, DH), jnp.float32),
            jax.ShapeDtypeStruct((NC, N, CW), jnp.float32),
        ],
        mesh=plsc.VectorSubcoreMesh(**_MESH),
        compiler_params=pltpu.CompilerParams(use_tc_tiling_on_sc=False),
        scratch_types=[
            pltpu.VMEM((2, IB, K), jnp.int32),
            pltpu.VMEM((2, IB, K), jnp.int32),
            pltpu.VMEM((K, DH), jnp.float32),
            pltpu.VMEM((K, DH), jnp.float32),
            pltpu.VMEM((K, DH), jnp.float32),
            pltpu.VMEM((K, DH), jnp.float32),
            pltpu.VMEM((ZCH, DH), jnp.float32),
            pltpu.VMEM((K, CW), jnp.float32),
            pltpu.VMEM((ZCH, CW), jnp.float32),
            pltpu.VMEM_SHARED((N, DH), jnp.float32),
            pltpu.VMEM_SHARED((N, CW), jnp.float32),
            pltpu.SemaphoreType.DMA,
            pltpu.SemaphoreType.DMA,
            pltpu.SemaphoreType.DMA,
            pltpu.SemaphoreType.DMA,
            pltpu.SemaphoreType.DMA,
        ],
    )(xa, xb, edges)


def _sc_agg(xa, xb, edges):
    return pl.kernel(
        _sc_agg_body,
        out_type=jax.ShapeDtypeStruct((NC, N, DH), jnp.float32),
        mesh=plsc.VectorSubcoreMesh(**_MESH),
        compiler_params=pltpu.CompilerParams(use_tc_tiling_on_sc=False),
        scratch_types=[
            pltpu.VMEM((2, IB, K), jnp.int32),
            pltpu.VMEM((2, IB, K), jnp.int32),
            pltpu.VMEM((K, DH), jnp.float32),
            pltpu.VMEM((K, DH), jnp.float32),
            pltpu.VMEM((K, DH), jnp.float32),
            pltpu.VMEM((K, DH), jnp.float32),
            pltpu.VMEM((ZCH, DH), jnp.float32),
            pltpu.VMEM_SHARED((N, DH), jnp.float32),
            pltpu.SemaphoreType.DMA,
            pltpu.SemaphoreType.DMA,
            pltpu.SemaphoreType.DMA,
            pltpu.SemaphoreType.DMA,
            pltpu.SemaphoreType.DMA,
        ],
    )(xa, xb, edges)


BN = 5000

_DNT = (((1,), (1,)), ((), ()))


def _matmul_t(a, w):
    return lax.dot_general(a, w, _DNT, preferred_element_type=jnp.float32)


def _tc_pre1_body(x_ref, wr_ref, b_ref, r_ref, xa_ref, xb_ref):
    x = x_ref[...]
    r_ref[...] = _matmul_t(x, wr_ref[...]) + b_ref[...]
    xa_ref[...] = x[:, :DH]
    xb_ref[...] = x[:, DH:]


def _tc_pre1(x, wr, b):
    return pl.pallas_call(
        _tc_pre1_body,
        grid=(N // BN,),
        in_specs=[
            pl.BlockSpec((BN, D), lambda i: (i, 0)),
            pl.BlockSpec((D, D), lambda i: (0, 0)),
            pl.BlockSpec((1, D), lambda i: (0, 0)),
        ],
        out_specs=[
            pl.BlockSpec((BN, D), lambda i: (i, 0)),
            pl.BlockSpec((BN, DH), lambda i: (i, 0)),
            pl.BlockSpec((BN, DH), lambda i: (i, 0)),
        ],
        out_shape=[
            jax.ShapeDtypeStruct((N, D), jnp.float32),
            jax.ShapeDtypeStruct((N, DH), jnp.float32),
            jax.ShapeDtypeStruct((N, DH), jnp.float32),
        ],
    )(x, wr, b)


def _tc_post1_body(sum_ref, cnt_ref, r_ref, wl_ref, wr2_ref, b2_ref,
                   ha_ref, hb_ref, r2_ref):
    ssum = jnp.concatenate([sum_ref[0], sum_ref[1]], axis=1)
    cnt = cnt_ref[0, :, 0:1] + cnt_ref[1, :, 0:1]
    inv = 1.0 / jnp.maximum(cnt, 1.0)
    mean = ssum * inv
    h = jnp.maximum(_matmul_t(mean, wl_ref[...]) + r_ref[...], 0.0)
    ha_ref[...] = h[:, :DH]
    hb_ref[...] = h[:, DH:]
    r2_ref[...] = _matmul_t(h, wr2_ref[...]) + b2_ref[...]


def _tc_post1(sum_p, cnt_p, r1, wl, wr2, b2):
    return pl.pallas_call(
        _tc_post1_body,
        grid=(N // BN,),
        in_specs=[
            pl.BlockSpec((NC, BN, DH), lambda i: (0, i, 0)),
            pl.BlockSpec((NC, BN, CW), lambda i: (0, i, 0)),
            pl.BlockSpec((BN, D), lambda i: (i, 0)),
            pl.BlockSpec((D, D), lambda i: (0, 0)),
            pl.BlockSpec((D, D), lambda i: (0, 0)),
            pl.BlockSpec((1, D), lambda i: (0, 0)),
        ],
        out_specs=[
            pl.BlockSpec((BN, DH), lambda i: (i, 0)),
            pl.BlockSpec((BN, DH), lambda i: (i, 0)),
            pl.BlockSpec((BN, D), lambda i: (i, 0)),
        ],
        out_shape=[
            jax.ShapeDtypeStruct((N, DH), jnp.float32),
            jax.ShapeDtypeStruct((N, DH), jnp.float32),
            jax.ShapeDtypeStruct((N, D), jnp.float32),
        ],
    )(sum_p, cnt_p, r1, wl, wr2, b2)


def _tc_post2_body(sum_ref, cnt_ref, r2_ref, wl_ref, wfc_ref, bfc_ref,
                   out_ref):
    ssum = jnp.concatenate([sum_ref[0], sum_ref[1]], axis=1)
    cnt = cnt_ref[0, :, 0:1] + cnt_ref[1, :, 0:1]
    inv = 1.0 / jnp.maximum(cnt, 1.0)
    mean = ssum * inv
    h2 = jnp.maximum(_matmul_t(mean, wl_ref[...]) + r2_ref[...], 0.0)
    logit = jnp.sum(h2 * wfc_ref[...], axis=1, keepdims=True)
    out_ref[...] = jax.nn.sigmoid(logit + bfc_ref[...])


def _tc_post2(sum_p, cnt_p, r2, wl, wfc, bfc):
    return pl.pallas_call(
        _tc_post2_body,
        grid=(N // BN,),
        in_specs=[
            pl.BlockSpec((NC, BN, DH), lambda i: (0, i, 0)),
            pl.BlockSpec((NC, BN, CW), lambda i: (0, i, 0)),
            pl.BlockSpec((BN, D), lambda i: (i, 0)),
            pl.BlockSpec((D, D), lambda i: (0, 0)),
            pl.BlockSpec((1, D), lambda i: (0, 0)),
            pl.BlockSpec((1, 1), lambda i: (0, 0)),
        ],
        out_specs=pl.BlockSpec((BN, 1), lambda i: (i, 0)),
        out_shape=jax.ShapeDtypeStruct((N, 1), jnp.float32),
    )(sum_p, cnt_p, r2, wl, wfc, bfc)


@jax.jit
def kernel(x, edge_index, W1_l, b1, W1_r, W2_l, b2, W2_r, Wfc, bfc):
    edges = edge_index.astype(jnp.int32).reshape(2, NS, NCHUNK, K)

    r1, xa, xb = _tc_pre1(x, W1_r, b1.reshape(1, D))
    sum1, cnt = _sc_agg_cnt(xa, xb, edges)
    ha, hb, r2 = _tc_post1(sum1, cnt, r1, W1_l, W2_r, b2.reshape(1, D))
    sum2 = _sc_agg(ha, hb, edges)
    out = _tc_post2(sum2, cnt, r2, W2_l, Wfc, bfc.reshape(1, 1))
    return out
